# baseline (speedup 1.0000x reference)
import functools

import jax
import jax.numpy as jnp
from jax import lax
from jax.experimental import pallas as pl
from jax.experimental.pallas import tpu as pltpu
from jax.experimental.pallas import tpu_sc as plsc

MAX_POS_ = 2048
HIDDEN_ = 768
BATCH_ = 4

NUM_CORES = 2
NUM_SUBCORES = 16
NUM_WORKERS = NUM_CORES * NUM_SUBCORES
P_SC = 256
CH_ROWS = P_SC // NUM_WORKERS
LANES = 16
NBUF = 3
NSTEP = BATCH_


def _sc_body(hid_hbm, pos_hbm, out_hbm,
             pos_v, buf0, buf1, buf2,
             psem, isem0, isem1, isem2, osem0, osem1, osem2):
    wid = lax.axis_index("s") * NUM_CORES + lax.axis_index("c")
    row_base = wid * CH_ROWS

    bufs = [buf0, buf1, buf2]
    isems = [isem0, isem1, isem2]
    osems = [osem0, osem1, osem2]

    def rows(ref):
        return ref.at[pl.ds(row_base, CH_ROWS)]

    pos_cp = pltpu.async_copy(rows(pos_hbm), pos_v, psem)

    in_h = [None] * NSTEP
    out_h = [None] * NSTEP
    for s in range(NBUF):
        in_h[s] = pltpu.async_copy(
            rows(hid_hbm.at[s]), bufs[s % NBUF], isems[s % NBUF])

    for s in range(NSTEP):
        if s == 0:
            pos_cp.wait()
        in_h[s].wait()
        hid_buf = bufs[s % NBUF]

        @plsc.parallel_loop(0, CH_ROWS, step=1)
        def _(r):
            for c in range(0, HIDDEN_, LANES):
                plsc.addupdate(hid_buf.at[r, pl.ds(c, LANES)],
                               pos_v[r, pl.ds(c, LANES)])

        out_h[s] = pltpu.async_copy(
            hid_buf, rows(out_hbm.at[s]), osems[s % NBUF])

        ns = s + 2
        if ns < NSTEP and in_h[ns] is None:
            if s >= 1:
                out_h[s - 1].wait()
            in_h[ns] = pltpu.async_copy(
                rows(hid_hbm.at[ns]), bufs[ns % NBUF], isems[ns % NBUF])

    out_h[NSTEP - 2].wait()
    out_h[NSTEP - 1].wait()


def _sc_part(hidden_states, pos_table):
    mesh = plsc.VectorSubcoreMesh(core_axis_name="c", subcore_axis_name="s")
    run = functools.partial(
        pl.kernel,
        mesh=mesh,
        out_type=jax.ShapeDtypeStruct((BATCH_, P_SC, HIDDEN_), jnp.float32),
        scratch_types=(
            [pltpu.VMEM((CH_ROWS, HIDDEN_), jnp.float32)
             for _ in range(1 + NBUF)]
            + [pltpu.SemaphoreType.DMA for _ in range(1 + 2 * NBUF)]
        ),
        compiler_params=pltpu.CompilerParams(use_tc_tiling_on_sc=True),
    )(_sc_body)
    return run(hidden_states, pos_table)


BP_TC = 256
N_TC_BLOCKS = (MAX_POS_ - P_SC) // BP_TC


def _tc_body(hid_ref, pos_ref, out_ref):
    out_ref[...] = hid_ref[...] + pos_ref[...]


def _tc_part(hidden_states, pos_table):
    grid = (N_TC_BLOCKS, 2)
    return pl.pallas_call(
        _tc_body,
        grid=grid,
        in_specs=[
            pl.BlockSpec((2, BP_TC, HIDDEN_), lambda i, j: (j, i + 1, 0)),
            pl.BlockSpec((BP_TC, HIDDEN_), lambda i, j: (i + 1, 0)),
        ],
        out_specs=pl.BlockSpec((2, BP_TC, HIDDEN_), lambda i, j: (j, i + 1, 0)),
        out_shape=jax.ShapeDtypeStruct((BATCH_, MAX_POS_, HIDDEN_), jnp.float32),
    )(hidden_states, pos_table)


def kernel(hidden_states, pos_table):
    tc_full = _tc_part(hidden_states, pos_table)
    sc_out = _sc_part(hidden_states, pos_table)
    return lax.dynamic_update_slice(tc_full, sc_out, (0, 0, 0))


# speedup vs baseline: 2.2249x; 2.2249x over previous
import jax
import jax.numpy as jnp
from jax.experimental import pallas as pl

MAX_POS_ = 2048
HIDDEN_ = 768
BATCH_ = 4


def _add_body(hid_ref, pos_ref, out_ref):
    out_ref[...] = hid_ref[...] + pos_ref[...]


def kernel(hidden_states, pos_table):
    grid = (2, 2)
    return pl.pallas_call(
        _add_body,
        grid=grid,
        in_specs=[
            pl.BlockSpec((2, 1024, HIDDEN_), lambda i, j: (j, i, 0)),
            pl.BlockSpec((1024, HIDDEN_), lambda i, j: (i, 0)),
        ],
        out_specs=pl.BlockSpec((2, 1024, HIDDEN_), lambda i, j: (j, i, 0)),
        out_shape=jax.ShapeDtypeStruct((BATCH_, MAX_POS_, HIDDEN_), jnp.float32),
    )(hidden_states, pos_table)


# speedup vs baseline: 2.3820x; 1.0706x over previous
import jax
import jax.numpy as jnp
from jax.experimental import pallas as pl

MAX_POS_ = 2048
HIDDEN_ = 768
BATCH_ = 4


def _add_body(hid_ref, pos_ref, out_ref):
    out_ref[...] = hid_ref[...] + pos_ref[...]


def kernel(hidden_states, pos_table):
    grid = (BATCH_ // 2,)
    return pl.pallas_call(
        _add_body,
        grid=grid,
        in_specs=[
            pl.BlockSpec((2, MAX_POS_, HIDDEN_), lambda i: (i, 0, 0)),
            pl.BlockSpec((MAX_POS_, HIDDEN_), lambda i: (0, 0)),
        ],
        out_specs=pl.BlockSpec((2, MAX_POS_, HIDDEN_), lambda i: (i, 0, 0)),
        out_shape=jax.ShapeDtypeStruct((BATCH_, MAX_POS_, HIDDEN_), jnp.float32),
    )(hidden_states, pos_table)
